# baseline (speedup 1.0000x reference)
import functools

import jax
import jax.numpy as jnp
from jax import lax
from jax.experimental import pallas as pl
from jax.experimental.pallas import tpu as pltpu
from jax.experimental.pallas import tpu_sc as plsc

_CHUNK = 16384
_NBUF = 2


def _build(n_atoms: int, n_species: int):
    info = plsc.get_sparse_core_info()
    nc, ns, nl = info.num_cores, info.num_subcores, info.num_lanes
    nw = nc * ns
    per_w = n_atoms // nw
    n_chunks = per_w // _CHUNK
    assert per_w * nw == n_atoms and n_chunks * _CHUNK == per_w

    mesh = plsc.VectorSubcoreMesh(core_axis_name="c", subcore_axis_name="s")

    @functools.partial(
        pl.kernel,
        mesh=mesh,
        out_type=jax.ShapeDtypeStruct((n_atoms,), jnp.float32),
        compiler_params=pltpu.CompilerParams(needs_layout_passes=False),
        scratch_types=[
            pltpu.VMEM((n_species,), jnp.float32),
            pltpu.VMEM_SHARED((_NBUF * 16 * _CHUNK,), jnp.int32),
            [pltpu.VMEM((_CHUNK,), jnp.int32) for _ in range(_NBUF)],
            [pltpu.VMEM((_CHUNK,), jnp.float32) for _ in range(_NBUF)],
            [pltpu.SemaphoreType.DMA for _ in range(_NBUF)],
            [pltpu.SemaphoreType.DMA for _ in range(_NBUF)],
            [pltpu.SemaphoreType.DMA for _ in range(_NBUF)],
        ],
    )
    def gather_kernel(species_hbm, const_hbm, out_hbm, table_v, idx_sh,
                      idx_bufs, out_bufs, in1_sems, in2_sems, out_sems):
        sid = lax.axis_index("s")
        wid = sid * nc + lax.axis_index("c")
        base = wid * per_w
        pltpu.sync_copy(const_hbm, table_v)

        def sh_slot(b):
            return idx_sh.at[pl.ds((b * ns + sid) * _CHUNK, _CHUNK)]

        def in1_copy(g, b):
            return pltpu.make_async_copy(
                species_hbm.at[pl.ds(base + g * _CHUNK, _CHUNK)],
                sh_slot(b), in1_sems[b])

        def in2_copy(b):
            return pltpu.make_async_copy(sh_slot(b), idx_bufs[b], in2_sems[b])

        def out_copy(g, b):
            return pltpu.make_async_copy(
                out_bufs[b], out_hbm.at[pl.ds(base + g * _CHUNK, _CHUNK)],
                out_sems[b])

        for b in range(min(_NBUF, n_chunks)):
            in1_copy(b, b).start()
        in1_copy(0, 0).wait()
        in2_copy(0).start()
        for g in range(n_chunks):
            b = g % _NBUF
            nb = (g + 1) % _NBUF
            if g + 1 < n_chunks:
                in1_copy(g + 1, nb).wait()
                in2_copy(nb).start()
            in2_copy(b).wait()
            if g >= _NBUF:
                out_copy(g - _NBUF, b).wait()
            idx_v, out_v = idx_bufs[b], out_bufs[b]

            @plsc.parallel_loop(0, _CHUNK, step=nl, unroll=8)
            def gather_body(i):
                iv = idx_v[pl.ds(i, nl)]
                out_v[pl.ds(i, nl)] = plsc.load_gather(table_v, [iv])

            out_copy(g, b).start()
            if g + _NBUF < n_chunks:
                in1_copy(g + _NBUF, b).start()
        for g in range(max(0, n_chunks - _NBUF), n_chunks):
            out_copy(g, g % _NBUF).wait()

    return gather_kernel


def kernel(species, constant):
    return _build(species.shape[0], constant.shape[0])(species, constant)


# speedup vs baseline: 1.2001x; 1.2001x over previous
import functools

import jax
import jax.numpy as jnp
from jax import lax
from jax.experimental import pallas as pl
from jax.experimental.pallas import tpu as pltpu
from jax.experimental.pallas import tpu_sc as plsc

_CHUNK = 16384
_NBUF = 2


def _build(n_atoms: int, n_species: int):
    info = plsc.get_sparse_core_info()
    nc, ns, nl = info.num_cores, info.num_subcores, info.num_lanes
    nw = nc * ns
    per_w = n_atoms // nw
    n_chunks = per_w // _CHUNK
    assert per_w * nw == n_atoms and n_chunks * _CHUNK == per_w

    mesh = plsc.VectorSubcoreMesh(core_axis_name="c", subcore_axis_name="s")

    @functools.partial(
        pl.kernel,
        mesh=mesh,
        out_type=jax.ShapeDtypeStruct((n_atoms,), jnp.float32),
        compiler_params=pltpu.CompilerParams(
            needs_layout_passes=False,
            disable_bounds_checks=True,
            disable_semaphore_checks=True,
            skip_device_barrier=True,
        ),
        scratch_types=[
            pltpu.VMEM((n_species,), jnp.float32),
            [pltpu.VMEM((_CHUNK,), jnp.int32) for _ in range(_NBUF)],
            [pltpu.VMEM((_CHUNK,), jnp.float32) for _ in range(_NBUF)],
            [pltpu.SemaphoreType.DMA for _ in range(_NBUF)],
            [pltpu.SemaphoreType.DMA for _ in range(_NBUF)],
        ],
    )
    def gather_kernel(species_hbm, const_hbm, out_hbm, table_v, idx_bufs,
                      out_bufs, in_sems, out_sems):
        wid = lax.axis_index("s") * nc + lax.axis_index("c")
        base = wid * per_w
        pltpu.sync_copy(const_hbm, table_v)

        def in_copy(g, b):
            return pltpu.make_async_copy(
                species_hbm.at[pl.ds(base + g * _CHUNK, _CHUNK)],
                idx_bufs[b], in_sems[b])

        def out_copy(g, b):
            return pltpu.make_async_copy(
                out_bufs[b], out_hbm.at[pl.ds(base + g * _CHUNK, _CHUNK)],
                out_sems[b])

        for b in range(min(_NBUF, n_chunks)):
            in_copy(b, b).start()
        for g in range(n_chunks):
            b = g % _NBUF
            in_copy(g, b).wait()
            if g >= _NBUF:
                out_copy(g - _NBUF, b).wait()
            idx_v, out_v = idx_bufs[b], out_bufs[b]

            @plsc.parallel_loop(0, _CHUNK, step=nl, unroll=8)
            def gather_body(i):
                iv = idx_v[pl.ds(i, nl)]
                out_v[pl.ds(i, nl)] = plsc.load_gather(table_v, [iv])

            out_copy(g, b).start()
            if g + _NBUF < n_chunks:
                in_copy(g + _NBUF, b).start()
        for g in range(max(0, n_chunks - _NBUF), n_chunks):
            out_copy(g, g % _NBUF).wait()

    return gather_kernel


def kernel(species, constant):
    return _build(species.shape[0], constant.shape[0])(species, constant)


# speedup vs baseline: 1.2405x; 1.0336x over previous
import functools

import jax
import jax.numpy as jnp
from jax import lax
from jax.experimental import pallas as pl
from jax.experimental.pallas import tpu as pltpu
from jax.experimental.pallas import tpu_sc as plsc

_CHUNK = 16384
_NBUF = 2


def _build(n_atoms: int, n_species: int):
    info = plsc.get_sparse_core_info()
    nc, ns, nl = info.num_cores, info.num_subcores, info.num_lanes
    nw = nc * ns
    per_w = n_atoms // nw
    n_chunks = per_w // _CHUNK
    assert per_w * nw == n_atoms and n_chunks * _CHUNK == per_w

    mesh = plsc.VectorSubcoreMesh(core_axis_name="c", subcore_axis_name="s")

    @functools.partial(
        pl.kernel,
        mesh=mesh,
        out_type=jax.ShapeDtypeStruct((n_atoms,), jnp.float32),
        compiler_params=pltpu.CompilerParams(
            needs_layout_passes=False,
            disable_bounds_checks=True,
            disable_semaphore_checks=True,
            skip_device_barrier=True,
        ),
        scratch_types=[
            pltpu.VMEM((n_species,), jnp.float32),
            [pltpu.VMEM((_CHUNK,), jnp.int32) for _ in range(_NBUF)],
            [pltpu.VMEM((_CHUNK,), jnp.float32) for _ in range(_NBUF)],
            [pltpu.SemaphoreType.DMA for _ in range(_NBUF)],
            [pltpu.SemaphoreType.DMA for _ in range(_NBUF)],
        ],
    )
    def gather_kernel(species_hbm, const_hbm, out_hbm, table_v, idx_bufs,
                      out_bufs, in_sems, out_sems):
        wid = lax.axis_index("s") * nc + lax.axis_index("c")
        base = wid * per_w

        def in_copy(g, b):
            return pltpu.make_async_copy(
                species_hbm.at[pl.ds(base + g * _CHUNK, _CHUNK)],
                idx_bufs[b], in_sems[b])

        def out_copy(g, b):
            return pltpu.make_async_copy(
                out_bufs[b], out_hbm.at[pl.ds(base + g * _CHUNK, _CHUNK)],
                out_sems[b])

        for b in range(min(_NBUF, n_chunks)):
            in_copy(b, b).start()
        pltpu.sync_copy(const_hbm, table_v)
        for g in range(n_chunks):
            b = g % _NBUF
            in_copy(g, b).wait()
            if g >= _NBUF:
                out_copy(g - _NBUF, b).wait()
            idx_v, out_v = idx_bufs[b], out_bufs[b]

            @plsc.parallel_loop(0, _CHUNK, step=nl, unroll=8)
            def gather_body(i):
                iv = idx_v[pl.ds(i, nl)]
                out_v[pl.ds(i, nl)] = plsc.load_gather(table_v, [iv])

            out_copy(g, b).start()
            if g + _NBUF < n_chunks:
                in_copy(g + _NBUF, b).start()
        for g in range(max(0, n_chunks - _NBUF), n_chunks):
            out_copy(g, g % _NBUF).wait()

    return gather_kernel


def kernel(species, constant):
    return _build(species.shape[0], constant.shape[0])(species, constant)
